# baseline (speedup 1.0000x reference)
import functools

import jax
import jax.numpy as jnp
import numpy as np
from jax import lax
from jax.experimental import pallas as pl
from jax.experimental.pallas import tpu as pltpu
from jax.experimental.pallas import tpu_sc as plsc

_GRID_Y, _GRID_X = 87, 116
_NB = 4
_SEG = _NB * _GRID_Y * _GRID_X
_SEGP = 40448
_SEGQ = _SEGP
_P = 6400
_NPAD = 1638400
_LO = -1e30
_HI = 1e30
_NW = 32


def _ka_body(n_real, p_ref, w1_ref, ids_ref, hpt_ref, s1_ref, sq1_ref):
    pid = pl.program_id(0)
    pts = p_ref[...]
    xi = jnp.floor(pts[:, 1] / 3.0).astype(jnp.int32)
    yi = jnp.floor(pts[:, 2] / 3.0).astype(jnp.int32)
    bid = pts[:, 0].astype(jnp.int32)
    ids = bid * (_GRID_Y * _GRID_X) + yi * _GRID_X + xi
    row = pid * _P + lax.broadcasted_iota(jnp.int32, (1, _P), 1)
    real = row < n_real
    ids = jnp.where(real, ids[None, :], _SEG)
    ids_ref[0] = ids
    x4 = pts[:, 1:5]
    hpt = lax.dot_general(w1_ref[...], x4, (((1,), (1,)), ((), ())),
                          preferred_element_type=jnp.float32)
    hpt_ref[...] = hpt
    m = real.astype(jnp.float32)

    @pl.when(pid == 0)
    def _():
        s1_ref[...] = jnp.zeros_like(s1_ref[...])
        sq1_ref[...] = jnp.zeros_like(sq1_ref[...])

    s1_ref[...] += jnp.sum(hpt * m, axis=1)[None, :]
    sq1_ref[...] += jnp.sum(hpt * hpt * m, axis=1)[None, :]


def _stage_a(points_pad, W1, n_real):
    nblk = _NPAD // _P
    return pl.pallas_call(
        functools.partial(_ka_body, n_real),
        grid=(nblk,),
        in_specs=[
            pl.BlockSpec((_P, 5), lambda i: (i, 0)),
            pl.BlockSpec((16, 4), lambda i: (0, 0)),
        ],
        out_specs=[
            pl.BlockSpec((1, 1, _P), lambda i: (i, 0, 0)),
            pl.BlockSpec((16, _P), lambda i: (0, i)),
            pl.BlockSpec((1, 16), lambda i: (0, 0)),
            pl.BlockSpec((1, 16), lambda i: (0, 0)),
        ],
        out_shape=[
            jax.ShapeDtypeStruct((nblk, 1, _P), jnp.int32),
            jax.ShapeDtypeStruct((16, _NPAD), jnp.float32),
            jax.ShapeDtypeStruct((1, 16), jnp.float32),
            jax.ShapeDtypeStruct((1, 16), jnp.float32),
        ],
    )(points_pad, W1)


def _sc_minmax_call(ids_flat, vals_t, n_cols, n_halves, n_rounds):
    n_tasks = n_cols * n_halves
    rng = _NPAD // n_halves
    K = 6400
    n_chunks = rng // K
    mesh = plsc.VectorSubcoreMesh(core_axis_name="c", subcore_axis_name="s", num_cores=2, num_subcores=16)

    def body(ids_hbm, vals_hbm, out_hbm, idbuf, vbuf, amin, amax):
        wid = lax.axis_index("c") * 16 + lax.axis_index("s")
        for r in range(n_rounds):
            task = wid + _NW * r
            col = lax.rem(task, n_cols)
            half = task // n_cols

            def init(i, _):
                amin[pl.ds(i * 16, 16)] = jnp.full((16,), _HI, jnp.float32)
                amax[pl.ds(i * 16, 16)] = jnp.full((16,), _LO, jnp.float32)
                return 0
            lax.fori_loop(0, _SEGP // 16, init, 0)

            def chunk(c, _):
                base = pl.multiple_of(half * rng + c * K, 256)
                voff = pl.multiple_of(col * _NPAD + base, 256)
                pltpu.sync_copy(ids_hbm.at[pl.ds(base, K)], idbuf)
                pltpu.sync_copy(vals_hbm.at[pl.ds(voff, K)], vbuf)

                lane = lax.iota(jnp.int32, 16)

                def rot(x, kk):
                    perm = lax.rem(lane + kk, 16)
                    return x.at[perm].get(mode="promise_in_bounds")

                def vec(j, _):
                    idx0 = idbuf[pl.ds(j * 16, 16)]
                    v0 = vbuf[pl.ds(j * 16, 16)]
                    sk, sv = plsc.sort_key_val(idx0, v0)
                    head = (sk != rot(sk, 15)) | (lane == 0)
                    vmax = sv
                    vmin = sv
                    for kk in (1, 2, 4, 8):
                        same = rot(sk, kk) == sk
                        vmax = jnp.where(
                            same, jnp.maximum(vmax, rot(vmax, kk)), vmax)
                        vmin = jnp.where(
                            same, jnp.minimum(vmin, rot(vmin, kk)), vmin)
                    cmx = plsc.load_gather(amax, [sk])
                    plsc.store_scatter(amax, [sk], jnp.maximum(cmx, vmax),
                                       mask=head)
                    cmn = plsc.load_gather(amin, [sk])
                    plsc.store_scatter(amin, [sk], jnp.minimum(cmn, vmin),
                                       mask=head)
                    return 0
                lax.fori_loop(0, K // 16, vec, 0)
                return 0
            lax.fori_loop(0, n_chunks, chunk, 0)
            omin = pl.multiple_of((task * 2) * _SEGP, 128)
            omax = pl.multiple_of((task * 2 + 1) * _SEGP, 128)
            pltpu.sync_copy(amin, out_hbm.at[pl.ds(omin, _SEGP)])
            pltpu.sync_copy(amax, out_hbm.at[pl.ds(omax, _SEGP)])

    k = pl.kernel(
        body,
        out_type=jax.ShapeDtypeStruct((n_tasks * 2 * _SEGP,), jnp.float32),
        mesh=mesh,
        compiler_params=pltpu.CompilerParams(needs_layout_passes=False),
        scratch_types=[
            pltpu.VMEM((K,), jnp.int32),
            pltpu.VMEM((K,), jnp.float32),
            pltpu.VMEM((_SEGP,), jnp.float32),
            pltpu.VMEM((_SEGP,), jnp.float32),
        ],
    )
    return k(ids_flat, vals_t)


def _sc_segsum_call(ids_flat, hpt_flat, sc1v, cc1v):
    K = 6400
    n_chunks = _NPAD // K
    mesh = plsc.VectorSubcoreMesh(core_axis_name="c", subcore_axis_name="s",
                                  num_cores=2, num_subcores=16)

    def body(ids_hbm, vals_hbm, sc_hbm, cc_hbm, out_hbm, idbuf, vbuf, pbuf, acc):
        wid = lax.axis_index("c") * 16 + lax.axis_index("s")

        @pl.when(wid < 17)
        def _():
            col = wid
            is_cnt = col >= 16
            colv = jnp.where(is_cnt, 0, col)
            pltpu.sync_copy(sc_hbm, pbuf.at[pl.ds(0, 16)])
            pltpu.sync_copy(cc_hbm, pbuf.at[pl.ds(16, 16)])

            def init(i, _):
                acc[pl.ds(i * 16, 16)] = jnp.zeros((16,), jnp.float32)
                return 0
            lax.fori_loop(0, _SEGP // 16, init, 0)

            def chunk(c, _):
                base = pl.multiple_of(c * K, 256)
                voff = pl.multiple_of(colv * _NPAD + base, 256)
                pltpu.sync_copy(ids_hbm.at[pl.ds(base, K)], idbuf)
                pltpu.sync_copy(vals_hbm.at[pl.ds(voff, K)], vbuf)

                lane = lax.iota(jnp.int32, 16)

                def rot(x, kk):
                    perm = lax.rem(lane + kk, 16)
                    return x.at[perm].get(mode="promise_in_bounds")

                cvec = jnp.full((16,), colv, jnp.int32)
                sca = pbuf[pl.ds(0, 16)].at[cvec].get(
                    mode="promise_in_bounds")
                cca = pbuf[pl.ds(16, 16)].at[cvec].get(
                    mode="promise_in_bounds")

                def vec(j, _):
                    idx0 = idbuf[pl.ds(j * 16, 16)]
                    hp = vbuf[pl.ds(j * 16, 16)]
                    z = hp * sca + cca
                    h = z / (1.0 + jnp.exp(-z))
                    v0 = jnp.where(is_cnt, 1.0, h)
                    sk, sv = plsc.sort_key_val(idx0, v0)
                    head = (sk != rot(sk, 15)) | (lane == 0)
                    vs = sv
                    for kk in (1, 2, 4, 8):
                        same = (rot(sk, kk) == sk) & (lane < 16 - kk)
                        vs = jnp.where(same, vs + rot(vs, kk), vs)
                    cur = plsc.load_gather(acc, [sk])
                    plsc.store_scatter(acc, [sk], cur + vs, mask=head)
                    return 0
                lax.fori_loop(0, K // 16, vec, 0)
                return 0
            lax.fori_loop(0, n_chunks, chunk, 0)
            ooff = pl.multiple_of(col * _SEGP, 128)
            pltpu.sync_copy(acc, out_hbm.at[pl.ds(ooff, _SEGP)])

    k = pl.kernel(
        body,
        out_type=jax.ShapeDtypeStruct((32 * _SEGP,), jnp.float32),
        mesh=mesh,
        compiler_params=pltpu.CompilerParams(needs_layout_passes=False),
        scratch_types=[
            pltpu.VMEM((K,), jnp.int32),
            pltpu.VMEM((K,), jnp.float32),
            pltpu.VMEM((32,), jnp.float32),
            pltpu.VMEM((_SEGP,), jnp.float32),
        ],
    )
    return k(ids_flat, hpt_flat, sc1v, cc1v)


def _kd_body(n_real, p_ref, w1_ref, w2a_ref, sc1_ref, cc1_ref,
             at_ref, sa_ref, sqa_ref):
    pid = pl.program_id(0)
    pts = p_ref[...]
    x4 = pts[:, 1:5]
    hp = lax.dot_general(x4, w1_ref[...], (((1,), (1,)), ((), ())),
                         preferred_element_type=jnp.float32)
    z = hp * sc1_ref[...] + cc1_ref[...]
    h = z * jax.nn.sigmoid(z)
    at = lax.dot_general(w2a_ref[...], h, (((1,), (1,)), ((), ())),
                         preferred_element_type=jnp.float32)
    at_ref[...] = at
    row = pid * _P + lax.broadcasted_iota(jnp.int32, (1, _P), 1)
    m = (row < n_real).astype(jnp.float32)

    @pl.when(pid == 0)
    def _():
        sa_ref[...] = jnp.zeros_like(sa_ref[...])
        sqa_ref[...] = jnp.zeros_like(sqa_ref[...])

    sa_ref[...] += jnp.sum(at * m, axis=1)[None, :]
    sqa_ref[...] += jnp.sum(at * at * m, axis=1)[None, :]


def _stage_d(points_pad, W1, W2a, sc1, cc1, n_real):
    nblk = _NPAD // _P
    return pl.pallas_call(
        functools.partial(_kd_body, n_real),
        grid=(nblk,),
        in_specs=[
            pl.BlockSpec((_P, 5), lambda i: (i, 0)),
            pl.BlockSpec((16, 4), lambda i: (0, 0)),
            pl.BlockSpec((64, 16), lambda i: (0, 0)),
            pl.BlockSpec((1, 16), lambda i: (0, 0)),
            pl.BlockSpec((1, 16), lambda i: (0, 0)),
        ],
        out_specs=[
            pl.BlockSpec((64, _P), lambda i: (0, i)),
            pl.BlockSpec((1, 64), lambda i: (0, 0)),
            pl.BlockSpec((1, 64), lambda i: (0, 0)),
        ],
        out_shape=[
            jax.ShapeDtypeStruct((64, _NPAD), jnp.float32),
            jax.ShapeDtypeStruct((1, 64), jnp.float32),
            jax.ShapeDtypeStruct((1, 64), jnp.float32),
        ],
    )(points_pad, W1, W2a, sc1, cc1)


def _kce_body(min1_ref, max1_ref, sw_ref, sc1_ref, cc1_ref, w2b_ref, w2a_ref,
              bseg_ref, mask_ref, c1_ref, c2_ref, c3_ref):
    pid = pl.program_id(0)
    sw = sw_ref[...]
    seg = pid * 128 + lax.broadcasted_iota(jnp.int32, (1, 128), 1)
    valid = seg < _SEG
    counts = jnp.where(valid, sw[16:17, :], 0.0)
    mask = (counts > 0).astype(jnp.float32)
    mask_ref[...] = mask
    za = min1_ref[...] * sc1_ref[...] + cc1_ref[...]
    zb = max1_ref[...] * sc1_ref[...] + cc1_ref[...]
    pa = za * jax.nn.sigmoid(za)
    pb = zb * jax.nn.sigmoid(zb)
    hmax = jnp.where(mask > 0, jnp.maximum(pa, pb), 0.0)
    bseg = lax.dot_general(w2b_ref[...], hmax, (((1,), (0,)), ((), ())),
                           preferred_element_type=jnp.float32)
    bseg_ref[...] = bseg
    sa = lax.dot_general(w2a_ref[...], sw[0:16, :],
                         (((1,), (0,)), ((), ())),
                         preferred_element_type=jnp.float32)

    @pl.when(pid == 0)
    def _():
        c1_ref[...] = jnp.zeros_like(c1_ref)
        c2_ref[...] = jnp.zeros_like(c2_ref)
        c3_ref[...] = jnp.zeros_like(c3_ref)

    c1_ref[...] += jnp.sum(bseg * counts, axis=1)[None, :]
    c2_ref[...] += jnp.sum(bseg * sa, axis=1)[None, :]
    c3_ref[...] += jnp.sum(bseg * bseg * counts, axis=1)[None, :]


def _stage_ce(min1t, max1t, swt, sc1b, cc1b, W2b, W2a):
    nblk = _SEGQ // 128
    return pl.pallas_call(
        _kce_body,
        grid=(nblk,),
        in_specs=[
            pl.BlockSpec((16, 128), lambda i: (0, i)),
            pl.BlockSpec((16, 128), lambda i: (0, i)),
            pl.BlockSpec((32, 128), lambda i: (0, i)),
            pl.BlockSpec((16, 128), lambda i: (0, 0)),
            pl.BlockSpec((16, 128), lambda i: (0, 0)),
            pl.BlockSpec((64, 16), lambda i: (0, 0)),
            pl.BlockSpec((64, 16), lambda i: (0, 0)),
        ],
        out_specs=[
            pl.BlockSpec((64, 128), lambda i: (0, i)),
            pl.BlockSpec((1, 128), lambda i: (0, i)),
            pl.BlockSpec((1, 64), lambda i: (0, 0)),
            pl.BlockSpec((1, 64), lambda i: (0, 0)),
            pl.BlockSpec((1, 64), lambda i: (0, 0)),
        ],
        out_shape=[
            jax.ShapeDtypeStruct((64, _SEGQ), jnp.float32),
            jax.ShapeDtypeStruct((1, _SEGQ), jnp.float32),
            jax.ShapeDtypeStruct((1, 64), jnp.float32),
            jax.ShapeDtypeStruct((1, 64), jnp.float32),
            jax.ShapeDtypeStruct((1, 64), jnp.float32),
        ],
    )(min1t, max1t, swt, sc1b, cc1b, W2b, W2a)


def _kg_body(min2_ref, max2_ref, bseg_ref, mask_ref, sc2_ref, cc2_ref, out_ref):
    b = bseg_ref[...]
    ua = (min2_ref[...] + b) * sc2_ref[...] + cc2_ref[...]
    ub = (max2_ref[...] + b) * sc2_ref[...] + cc2_ref[...]
    fa = ua * jax.nn.sigmoid(ua)
    fb = ub * jax.nn.sigmoid(ub)
    out_ref[...] = jnp.where(mask_ref[...] > 0, jnp.maximum(fa, fb), 0.0)


def _stage_g(min2t, max2t, bsegt, maskt, sc2b, cc2b):
    nblk = _SEGQ // 128
    return pl.pallas_call(
        _kg_body,
        grid=(nblk,),
        in_specs=[
            pl.BlockSpec((64, 128), lambda i: (0, i)),
            pl.BlockSpec((64, 128), lambda i: (0, i)),
            pl.BlockSpec((64, 128), lambda i: (0, i)),
            pl.BlockSpec((1, 128), lambda i: (0, i)),
            pl.BlockSpec((64, 128), lambda i: (0, 0)),
            pl.BlockSpec((64, 128), lambda i: (0, 0)),
        ],
        out_specs=pl.BlockSpec((64, 128), lambda i: (0, i)),
        out_shape=jax.ShapeDtypeStruct((64, _SEGQ), jnp.float32),
    )(min2t, max2t, bsegt, maskt, sc2b, cc2b)


def _interp_matrix(n_in, n_out):
    s = np.linspace(0.0, n_in - 1.0, n_out)
    i0 = np.floor(s).astype(np.int32)
    i1 = np.clip(i0 + 1, 0, n_in - 1)
    w = (s - i0).astype(np.float32)
    m = np.zeros((n_out, n_in), dtype=np.float32)
    m[np.arange(n_out), i0] += 1.0 - w
    m[np.arange(n_out), i1] += w
    return m


def _bilinear_kernel(d_ref, ay_ref, axt_ref, o_ref):
    t = jnp.dot(ay_ref[...], d_ref[0], preferred_element_type=jnp.float32)
    o_ref[0] = jnp.dot(t, axt_ref[...], preferred_element_type=jnp.float32)


def _bilinear_upsample(dense_bc):
    ay = jnp.asarray(_interp_matrix(_GRID_Y, 224))
    axt = jnp.asarray(_interp_matrix(_GRID_X, 224).T)
    n = dense_bc.shape[0]
    return pl.pallas_call(
        _bilinear_kernel,
        grid=(n,),
        in_specs=[
            pl.BlockSpec((1, _GRID_Y, _GRID_X), lambda i: (i, 0, 0)),
            pl.BlockSpec((224, _GRID_Y), lambda i: (0, 0)),
            pl.BlockSpec((_GRID_X, 224), lambda i: (0, 0)),
        ],
        out_specs=pl.BlockSpec((1, 224, 224), lambda i: (i, 0, 0)),
        out_shape=jax.ShapeDtypeStruct((n, 224, 224), jnp.float32),
    )(dense_bc, ay, axt)


def kernel(fus, points, W1, g1, b1, W2, g2, b2):
    n_real = points.shape[0]
    points_pad = jnp.concatenate(
        [points, jnp.zeros((_NPAD - n_real, 5), jnp.float32)], axis=0)

    inv = jnp.asarray([1 / 346.0, 1 / 260.0, 1 / 200.0, 1.0], jnp.float32)
    W1s = W1 * inv[None, :]
    ids3, hpt, s1, sq1 = _stage_a(points_pad, W1s, n_real)
    ids_flat = ids3.reshape(_NPAD)

    mean1 = s1[0] / n_real
    var1 = sq1[0] / n_real - mean1 * mean1
    sc1 = (g1 / jnp.sqrt(var1 + 1e-3))[None, :]
    cc1 = (b1 - mean1 * sc1[0])[None, :]

    mm1 = _sc_minmax_call(ids_flat, hpt.reshape(16 * _NPAD), 16, 2, 1)
    mm1 = mm1.reshape(2, 16, 2, _SEGP)
    min1 = jnp.minimum(mm1[0, :, 0], mm1[1, :, 0])
    max1 = jnp.maximum(mm1[0, :, 1], mm1[1, :, 1])

    W2a, W2b = W2[:, :16], W2[:, 16:]
    at, sa, sqa = _stage_d(points_pad, W1s, W2a, sc1, cc1, n_real)

    ssum = _sc_segsum_call(ids_flat, hpt.reshape(16 * _NPAD), sc1[0], cc1[0])

    mm2 = _sc_minmax_call(ids_flat, at.reshape(64 * _NPAD), 64, 1, 2)
    mm2 = mm2.reshape(64, 2, _SEGP)

    min1t = min1
    max1t = max1
    swt = ssum.reshape(32, _SEGP)
    sc1b = jnp.broadcast_to(sc1.T, (16, 128))
    cc1b = jnp.broadcast_to(cc1.T, (16, 128))
    bsegt, maskt, c1, c2, c3 = _stage_ce(min1t, max1t, swt, sc1b, cc1b, W2b, W2a)

    mean2 = (sa[0] + c1[0]) / n_real
    ex2 = (sqa[0] + 2.0 * c2[0] + c3[0]) / n_real
    var2 = ex2 - mean2 * mean2
    sc2 = g2 / jnp.sqrt(var2 + 1e-3)
    cc2 = b2 - mean2 * sc2
    sc2b = jnp.broadcast_to(sc2[:, None], (64, 128))
    cc2b = jnp.broadcast_to(cc2[:, None], (64, 128))

    min2t = mm2[:, 0]
    max2t = mm2[:, 1]

    featt = _stage_g(min2t, max2t, bsegt, maskt, sc2b, cc2b)

    feat = featt[:, :_SEG].reshape(64, _NB, _GRID_Y, _GRID_X)
    dense = jnp.transpose(feat, (1, 0, 2, 3)).reshape(_NB * 64, _GRID_Y, _GRID_X)
    out = _bilinear_upsample(dense)
    return out.reshape(_NB, 64, 224, 224)


# speedup vs baseline: 1.0169x; 1.0169x over previous
import functools

import jax
import jax.numpy as jnp
import numpy as np
from jax import lax
from jax.experimental import pallas as pl
from jax.experimental.pallas import tpu as pltpu
from jax.experimental.pallas import tpu_sc as plsc

_GRID_Y, _GRID_X = 87, 116
_NB = 4
_SEG = _NB * _GRID_Y * _GRID_X
_SEGP = 40448
_SEGQ = _SEGP
_P = 6400
_NPAD = 1638400
_LO = -1e30
_HI = 1e30
_NW = 32


def _ka_body(n_real, p_ref, w1_ref, ids_ref, hpt_ref, s1_ref, sq1_ref):
    pid = pl.program_id(0)
    pts = p_ref[...]
    xi = jnp.floor(pts[:, 1] / 3.0).astype(jnp.int32)
    yi = jnp.floor(pts[:, 2] / 3.0).astype(jnp.int32)
    bid = pts[:, 0].astype(jnp.int32)
    ids = bid * (_GRID_Y * _GRID_X) + yi * _GRID_X + xi
    row = pid * _P + lax.broadcasted_iota(jnp.int32, (1, _P), 1)
    real = row < n_real
    ids = jnp.where(real, ids[None, :], _SEG)
    ids_ref[0] = ids
    x4 = pts[:, 1:5]
    hpt = lax.dot_general(w1_ref[...], x4, (((1,), (1,)), ((), ())),
                          preferred_element_type=jnp.float32)
    hpt_ref[...] = hpt
    m = real.astype(jnp.float32)

    @pl.when(pid == 0)
    def _():
        s1_ref[...] = jnp.zeros_like(s1_ref[...])
        sq1_ref[...] = jnp.zeros_like(sq1_ref[...])

    s1_ref[...] += jnp.sum(hpt * m, axis=1)[None, :]
    sq1_ref[...] += jnp.sum(hpt * hpt * m, axis=1)[None, :]


def _stage_a(points_pad, W1, n_real):
    nblk = _NPAD // _P
    return pl.pallas_call(
        functools.partial(_ka_body, n_real),
        grid=(nblk,),
        in_specs=[
            pl.BlockSpec((_P, 5), lambda i: (i, 0)),
            pl.BlockSpec((16, 4), lambda i: (0, 0)),
        ],
        out_specs=[
            pl.BlockSpec((1, 1, _P), lambda i: (i, 0, 0)),
            pl.BlockSpec((16, _P), lambda i: (0, i)),
            pl.BlockSpec((1, 16), lambda i: (0, 0)),
            pl.BlockSpec((1, 16), lambda i: (0, 0)),
        ],
        out_shape=[
            jax.ShapeDtypeStruct((nblk, 1, _P), jnp.int32),
            jax.ShapeDtypeStruct((16, _NPAD), jnp.float32),
            jax.ShapeDtypeStruct((1, 16), jnp.float32),
            jax.ShapeDtypeStruct((1, 16), jnp.float32),
        ],
    )(points_pad, W1)


def _sc_minmax_call(ids_flat, vals_t, n_cols, n_halves, n_rounds):
    n_tasks = n_cols * n_halves
    rng = _NPAD // n_halves
    K = 6400
    n_chunks = rng // K
    mesh = plsc.VectorSubcoreMesh(core_axis_name="c", subcore_axis_name="s", num_cores=2, num_subcores=16)

    def body(ids_hbm, vals_hbm, out_hbm, idbuf, vbuf, amin, amax):
        wid = lax.axis_index("c") * 16 + lax.axis_index("s")
        for r in range(n_rounds):
            task = wid + _NW * r
            col = lax.rem(task, n_cols)
            half = task // n_cols

            def init(i, _):
                amin[pl.ds(i * 16, 16)] = jnp.full((16,), _HI, jnp.float32)
                amax[pl.ds(i * 16, 16)] = jnp.full((16,), _LO, jnp.float32)
                return 0
            lax.fori_loop(0, _SEGP // 16, init, 0)

            def chunk(c, _):
                base = pl.multiple_of(half * rng + c * K, 256)
                voff = pl.multiple_of(col * _NPAD + base, 256)
                pltpu.sync_copy(ids_hbm.at[pl.ds(base, K)], idbuf)
                pltpu.sync_copy(vals_hbm.at[pl.ds(voff, K)], vbuf)

                lane = lax.iota(jnp.int32, 16)

                def rot(x, kk):
                    perm = lax.rem(lane + kk, 16)
                    return x.at[perm].get(mode="promise_in_bounds")

                def vec1(j):
                    idx0 = idbuf[pl.ds(j * 16, 16)]
                    v0 = vbuf[pl.ds(j * 16, 16)]
                    sk, sv = plsc.sort_key_val(idx0, v0)
                    head = (sk != rot(sk, 15)) | (lane == 0)
                    vmax = sv
                    vmin = sv
                    for kk in (1, 2, 4, 8):
                        same = rot(sk, kk) == sk
                        vmax = jnp.where(
                            same, jnp.maximum(vmax, rot(vmax, kk)), vmax)
                        vmin = jnp.where(
                            same, jnp.minimum(vmin, rot(vmin, kk)), vmin)
                    cmx = plsc.load_gather(amax, [sk])
                    plsc.store_scatter(amax, [sk], jnp.maximum(cmx, vmax),
                                       mask=head)
                    cmn = plsc.load_gather(amin, [sk])
                    plsc.store_scatter(amin, [sk], jnp.minimum(cmn, vmin),
                                       mask=head)

                def vec(j, _):
                    for u in range(4):
                        vec1(j * 4 + u)
                    return 0
                lax.fori_loop(0, K // 64, vec, 0)
                return 0
            lax.fori_loop(0, n_chunks, chunk, 0)
            omin = pl.multiple_of((task * 2) * _SEGP, 128)
            omax = pl.multiple_of((task * 2 + 1) * _SEGP, 128)
            pltpu.sync_copy(amin, out_hbm.at[pl.ds(omin, _SEGP)])
            pltpu.sync_copy(amax, out_hbm.at[pl.ds(omax, _SEGP)])

    k = pl.kernel(
        body,
        out_type=jax.ShapeDtypeStruct((n_tasks * 2 * _SEGP,), jnp.float32),
        mesh=mesh,
        compiler_params=pltpu.CompilerParams(needs_layout_passes=False),
        scratch_types=[
            pltpu.VMEM((K,), jnp.int32),
            pltpu.VMEM((K,), jnp.float32),
            pltpu.VMEM((_SEGP,), jnp.float32),
            pltpu.VMEM((_SEGP,), jnp.float32),
        ],
    )
    return k(ids_flat, vals_t)


def _sc_segsum_call(ids_flat, hpt_flat, sc1v, cc1v):
    K = 6400
    n_chunks = _NPAD // K
    mesh = plsc.VectorSubcoreMesh(core_axis_name="c", subcore_axis_name="s",
                                  num_cores=2, num_subcores=16)

    def body(ids_hbm, vals_hbm, sc_hbm, cc_hbm, out_hbm, idbuf, vbuf, pbuf, acc):
        wid = lax.axis_index("c") * 16 + lax.axis_index("s")

        @pl.when(wid < 17)
        def _():
            col = wid
            is_cnt = col >= 16
            colv = jnp.where(is_cnt, 0, col)
            pltpu.sync_copy(sc_hbm, pbuf.at[pl.ds(0, 16)])
            pltpu.sync_copy(cc_hbm, pbuf.at[pl.ds(16, 16)])

            def init(i, _):
                acc[pl.ds(i * 16, 16)] = jnp.zeros((16,), jnp.float32)
                return 0
            lax.fori_loop(0, _SEGP // 16, init, 0)

            def chunk(c, _):
                base = pl.multiple_of(c * K, 256)
                voff = pl.multiple_of(colv * _NPAD + base, 256)
                pltpu.sync_copy(ids_hbm.at[pl.ds(base, K)], idbuf)
                pltpu.sync_copy(vals_hbm.at[pl.ds(voff, K)], vbuf)

                lane = lax.iota(jnp.int32, 16)

                def rot(x, kk):
                    perm = lax.rem(lane + kk, 16)
                    return x.at[perm].get(mode="promise_in_bounds")

                cvec = jnp.full((16,), colv, jnp.int32)
                sca = pbuf[pl.ds(0, 16)].at[cvec].get(
                    mode="promise_in_bounds")
                cca = pbuf[pl.ds(16, 16)].at[cvec].get(
                    mode="promise_in_bounds")

                def vec1(j):
                    idx0 = idbuf[pl.ds(j * 16, 16)]
                    hp = vbuf[pl.ds(j * 16, 16)]
                    z = hp * sca + cca
                    h = z / (1.0 + jnp.exp(-z))
                    v0 = jnp.where(is_cnt, 1.0, h)
                    sk, sv = plsc.sort_key_val(idx0, v0)
                    head = (sk != rot(sk, 15)) | (lane == 0)
                    vs = sv
                    for kk in (1, 2, 4, 8):
                        same = (rot(sk, kk) == sk) & (lane < 16 - kk)
                        vs = jnp.where(same, vs + rot(vs, kk), vs)
                    cur = plsc.load_gather(acc, [sk])
                    plsc.store_scatter(acc, [sk], cur + vs, mask=head)

                def vec(j, _):
                    for u in range(4):
                        vec1(j * 4 + u)
                    return 0
                lax.fori_loop(0, K // 64, vec, 0)
                return 0
            lax.fori_loop(0, n_chunks, chunk, 0)
            ooff = pl.multiple_of(col * _SEGP, 128)
            pltpu.sync_copy(acc, out_hbm.at[pl.ds(ooff, _SEGP)])

    k = pl.kernel(
        body,
        out_type=jax.ShapeDtypeStruct((32 * _SEGP,), jnp.float32),
        mesh=mesh,
        compiler_params=pltpu.CompilerParams(needs_layout_passes=False),
        scratch_types=[
            pltpu.VMEM((K,), jnp.int32),
            pltpu.VMEM((K,), jnp.float32),
            pltpu.VMEM((32,), jnp.float32),
            pltpu.VMEM((_SEGP,), jnp.float32),
        ],
    )
    return k(ids_flat, hpt_flat, sc1v, cc1v)


def _kd_body(n_real, p_ref, w1_ref, w2a_ref, sc1_ref, cc1_ref,
             at_ref, sa_ref, sqa_ref):
    pid = pl.program_id(0)
    pts = p_ref[...]
    x4 = pts[:, 1:5]
    hp = lax.dot_general(x4, w1_ref[...], (((1,), (1,)), ((), ())),
                         preferred_element_type=jnp.float32)
    z = hp * sc1_ref[...] + cc1_ref[...]
    h = z * jax.nn.sigmoid(z)
    at = lax.dot_general(w2a_ref[...], h, (((1,), (1,)), ((), ())),
                         preferred_element_type=jnp.float32)
    at_ref[...] = at
    row = pid * _P + lax.broadcasted_iota(jnp.int32, (1, _P), 1)
    m = (row < n_real).astype(jnp.float32)

    @pl.when(pid == 0)
    def _():
        sa_ref[...] = jnp.zeros_like(sa_ref[...])
        sqa_ref[...] = jnp.zeros_like(sqa_ref[...])

    sa_ref[...] += jnp.sum(at * m, axis=1)[None, :]
    sqa_ref[...] += jnp.sum(at * at * m, axis=1)[None, :]


def _stage_d(points_pad, W1, W2a, sc1, cc1, n_real):
    nblk = _NPAD // _P
    return pl.pallas_call(
        functools.partial(_kd_body, n_real),
        grid=(nblk,),
        in_specs=[
            pl.BlockSpec((_P, 5), lambda i: (i, 0)),
            pl.BlockSpec((16, 4), lambda i: (0, 0)),
            pl.BlockSpec((64, 16), lambda i: (0, 0)),
            pl.BlockSpec((1, 16), lambda i: (0, 0)),
            pl.BlockSpec((1, 16), lambda i: (0, 0)),
        ],
        out_specs=[
            pl.BlockSpec((64, _P), lambda i: (0, i)),
            pl.BlockSpec((1, 64), lambda i: (0, 0)),
            pl.BlockSpec((1, 64), lambda i: (0, 0)),
        ],
        out_shape=[
            jax.ShapeDtypeStruct((64, _NPAD), jnp.float32),
            jax.ShapeDtypeStruct((1, 64), jnp.float32),
            jax.ShapeDtypeStruct((1, 64), jnp.float32),
        ],
    )(points_pad, W1, W2a, sc1, cc1)


def _kce_body(min1_ref, max1_ref, sw_ref, sc1_ref, cc1_ref, w2b_ref, w2a_ref,
              bseg_ref, mask_ref, c1_ref, c2_ref, c3_ref):
    pid = pl.program_id(0)
    sw = sw_ref[...]
    seg = pid * 128 + lax.broadcasted_iota(jnp.int32, (1, 128), 1)
    valid = seg < _SEG
    counts = jnp.where(valid, sw[16:17, :], 0.0)
    mask = (counts > 0).astype(jnp.float32)
    mask_ref[...] = mask
    za = min1_ref[...] * sc1_ref[...] + cc1_ref[...]
    zb = max1_ref[...] * sc1_ref[...] + cc1_ref[...]
    pa = za * jax.nn.sigmoid(za)
    pb = zb * jax.nn.sigmoid(zb)
    hmax = jnp.where(mask > 0, jnp.maximum(pa, pb), 0.0)
    bseg = lax.dot_general(w2b_ref[...], hmax, (((1,), (0,)), ((), ())),
                           preferred_element_type=jnp.float32)
    bseg_ref[...] = bseg
    sa = lax.dot_general(w2a_ref[...], sw[0:16, :],
                         (((1,), (0,)), ((), ())),
                         preferred_element_type=jnp.float32)

    @pl.when(pid == 0)
    def _():
        c1_ref[...] = jnp.zeros_like(c1_ref)
        c2_ref[...] = jnp.zeros_like(c2_ref)
        c3_ref[...] = jnp.zeros_like(c3_ref)

    c1_ref[...] += jnp.sum(bseg * counts, axis=1)[None, :]
    c2_ref[...] += jnp.sum(bseg * sa, axis=1)[None, :]
    c3_ref[...] += jnp.sum(bseg * bseg * counts, axis=1)[None, :]


def _stage_ce(min1t, max1t, swt, sc1b, cc1b, W2b, W2a):
    nblk = _SEGQ // 128
    return pl.pallas_call(
        _kce_body,
        grid=(nblk,),
        in_specs=[
            pl.BlockSpec((16, 128), lambda i: (0, i)),
            pl.BlockSpec((16, 128), lambda i: (0, i)),
            pl.BlockSpec((32, 128), lambda i: (0, i)),
            pl.BlockSpec((16, 128), lambda i: (0, 0)),
            pl.BlockSpec((16, 128), lambda i: (0, 0)),
            pl.BlockSpec((64, 16), lambda i: (0, 0)),
            pl.BlockSpec((64, 16), lambda i: (0, 0)),
        ],
        out_specs=[
            pl.BlockSpec((64, 128), lambda i: (0, i)),
            pl.BlockSpec((1, 128), lambda i: (0, i)),
            pl.BlockSpec((1, 64), lambda i: (0, 0)),
            pl.BlockSpec((1, 64), lambda i: (0, 0)),
            pl.BlockSpec((1, 64), lambda i: (0, 0)),
        ],
        out_shape=[
            jax.ShapeDtypeStruct((64, _SEGQ), jnp.float32),
            jax.ShapeDtypeStruct((1, _SEGQ), jnp.float32),
            jax.ShapeDtypeStruct((1, 64), jnp.float32),
            jax.ShapeDtypeStruct((1, 64), jnp.float32),
            jax.ShapeDtypeStruct((1, 64), jnp.float32),
        ],
    )(min1t, max1t, swt, sc1b, cc1b, W2b, W2a)


def _kg_body(min2_ref, max2_ref, bseg_ref, mask_ref, sc2_ref, cc2_ref, out_ref):
    b = bseg_ref[...]
    ua = (min2_ref[...] + b) * sc2_ref[...] + cc2_ref[...]
    ub = (max2_ref[...] + b) * sc2_ref[...] + cc2_ref[...]
    fa = ua * jax.nn.sigmoid(ua)
    fb = ub * jax.nn.sigmoid(ub)
    out_ref[...] = jnp.where(mask_ref[...] > 0, jnp.maximum(fa, fb), 0.0)


def _stage_g(min2t, max2t, bsegt, maskt, sc2b, cc2b):
    nblk = _SEGQ // 128
    return pl.pallas_call(
        _kg_body,
        grid=(nblk,),
        in_specs=[
            pl.BlockSpec((64, 128), lambda i: (0, i)),
            pl.BlockSpec((64, 128), lambda i: (0, i)),
            pl.BlockSpec((64, 128), lambda i: (0, i)),
            pl.BlockSpec((1, 128), lambda i: (0, i)),
            pl.BlockSpec((64, 128), lambda i: (0, 0)),
            pl.BlockSpec((64, 128), lambda i: (0, 0)),
        ],
        out_specs=pl.BlockSpec((64, 128), lambda i: (0, i)),
        out_shape=jax.ShapeDtypeStruct((64, _SEGQ), jnp.float32),
    )(min2t, max2t, bsegt, maskt, sc2b, cc2b)


def _interp_matrix(n_in, n_out):
    s = np.linspace(0.0, n_in - 1.0, n_out)
    i0 = np.floor(s).astype(np.int32)
    i1 = np.clip(i0 + 1, 0, n_in - 1)
    w = (s - i0).astype(np.float32)
    m = np.zeros((n_out, n_in), dtype=np.float32)
    m[np.arange(n_out), i0] += 1.0 - w
    m[np.arange(n_out), i1] += w
    return m


def _bilinear_kernel(d_ref, ay_ref, axt_ref, o_ref):
    t = jnp.dot(ay_ref[...], d_ref[0], preferred_element_type=jnp.float32)
    o_ref[0] = jnp.dot(t, axt_ref[...], preferred_element_type=jnp.float32)


def _bilinear_upsample(dense_bc):
    ay = jnp.asarray(_interp_matrix(_GRID_Y, 224))
    axt = jnp.asarray(_interp_matrix(_GRID_X, 224).T)
    n = dense_bc.shape[0]
    return pl.pallas_call(
        _bilinear_kernel,
        grid=(n,),
        in_specs=[
            pl.BlockSpec((1, _GRID_Y, _GRID_X), lambda i: (i, 0, 0)),
            pl.BlockSpec((224, _GRID_Y), lambda i: (0, 0)),
            pl.BlockSpec((_GRID_X, 224), lambda i: (0, 0)),
        ],
        out_specs=pl.BlockSpec((1, 224, 224), lambda i: (i, 0, 0)),
        out_shape=jax.ShapeDtypeStruct((n, 224, 224), jnp.float32),
    )(dense_bc, ay, axt)


def kernel(fus, points, W1, g1, b1, W2, g2, b2):
    n_real = points.shape[0]
    points_pad = jnp.concatenate(
        [points, jnp.zeros((_NPAD - n_real, 5), jnp.float32)], axis=0)

    inv = jnp.asarray([1 / 346.0, 1 / 260.0, 1 / 200.0, 1.0], jnp.float32)
    W1s = W1 * inv[None, :]
    ids3, hpt, s1, sq1 = _stage_a(points_pad, W1s, n_real)
    ids_flat = ids3.reshape(_NPAD)

    mean1 = s1[0] / n_real
    var1 = sq1[0] / n_real - mean1 * mean1
    sc1 = (g1 / jnp.sqrt(var1 + 1e-3))[None, :]
    cc1 = (b1 - mean1 * sc1[0])[None, :]

    mm1 = _sc_minmax_call(ids_flat, hpt.reshape(16 * _NPAD), 16, 2, 1)
    mm1 = mm1.reshape(2, 16, 2, _SEGP)
    min1 = jnp.minimum(mm1[0, :, 0], mm1[1, :, 0])
    max1 = jnp.maximum(mm1[0, :, 1], mm1[1, :, 1])

    W2a, W2b = W2[:, :16], W2[:, 16:]
    at, sa, sqa = _stage_d(points_pad, W1s, W2a, sc1, cc1, n_real)

    ssum = _sc_segsum_call(ids_flat, hpt.reshape(16 * _NPAD), sc1[0], cc1[0])

    mm2 = _sc_minmax_call(ids_flat, at.reshape(64 * _NPAD), 64, 1, 2)
    mm2 = mm2.reshape(64, 2, _SEGP)

    min1t = min1
    max1t = max1
    swt = ssum.reshape(32, _SEGP)
    sc1b = jnp.broadcast_to(sc1.T, (16, 128))
    cc1b = jnp.broadcast_to(cc1.T, (16, 128))
    bsegt, maskt, c1, c2, c3 = _stage_ce(min1t, max1t, swt, sc1b, cc1b, W2b, W2a)

    mean2 = (sa[0] + c1[0]) / n_real
    ex2 = (sqa[0] + 2.0 * c2[0] + c3[0]) / n_real
    var2 = ex2 - mean2 * mean2
    sc2 = g2 / jnp.sqrt(var2 + 1e-3)
    cc2 = b2 - mean2 * sc2
    sc2b = jnp.broadcast_to(sc2[:, None], (64, 128))
    cc2b = jnp.broadcast_to(cc2[:, None], (64, 128))

    min2t = mm2[:, 0]
    max2t = mm2[:, 1]

    featt = _stage_g(min2t, max2t, bsegt, maskt, sc2b, cc2b)

    feat = featt[:, :_SEG].reshape(64, _NB, _GRID_Y, _GRID_X)
    dense = jnp.transpose(feat, (1, 0, 2, 3)).reshape(_NB * 64, _GRID_Y, _GRID_X)
    out = _bilinear_upsample(dense)
    return out.reshape(_NB, 64, 224, 224)
